# baseline (speedup 1.0000x reference)
import functools

import jax
import jax.numpy as jnp
from jax import lax
from jax.experimental import pallas as pl
from jax.experimental.pallas import tpu as pltpu
from jax.experimental.pallas import tpu_sc as plsc

N_NODES = 10000
N_EDGES = 320000
D = 128
N_GRAPHS = 64

NC = 2
NS = 16
NW = NC * NS

CHUNK = 128
IB = 8
_CPT_MIN = (N_EDGES + NW * CHUNK - 1) // (NW * CHUNK)
CPT = ((_CPT_MIN + 2 * IB - 1) // (2 * IB)) * (2 * IB)
NBLK = CPT // IB
E_PAD = NW * CPT * CHUNK

ACC_ROWS = 10240
ZERO_ROWS_PER_TILE = ACC_ROWS // NS


def _segment_sum_sc(x, src_t, dst_t):
    mesh = plsc.VectorSubcoreMesh(core_axis_name="c", subcore_axis_name="s")

    @functools.partial(
        pl.kernel,
        out_type=jax.ShapeDtypeStruct((NC, ACC_ROWS, D), jnp.float32),
        mesh=mesh,
        scratch_types=[
            pltpu.VMEM_SHARED((ACC_ROWS, D), jnp.float32),
            pltpu.VMEM((2, IB, CHUNK), jnp.int32),
            pltpu.VMEM((2, IB, CHUNK), jnp.int32),
            pltpu.VMEM((CHUNK, D), jnp.float32),
            pltpu.VMEM((CHUNK, D), jnp.float32),
            pltpu.SemaphoreType.DMA,
            pltpu.SemaphoreType.DMA,
            pltpu.SemaphoreType.DMA,
            pltpu.SemaphoreType.DMA,
        ],
    )
    def seg_sum(x_hbm, src_hbm, dst_hbm, out_hbm, acc, sblk, dblk, buf0, buf1,
                gsem0, gsem1, isem0, isem1):
        c = lax.axis_index("c")
        s = lax.axis_index("s")
        wid = c * NS + s

        zvec = jnp.zeros((16,), jnp.float32)

        @pl.loop(0, CHUNK)
        def _zero_rows(r):
            for cc in range(D // 16):
                buf0[r, pl.ds(cc * 16, 16)] = zvec

        off = 0
        while off < ZERO_ROWS_PER_TILE:
            n = min(CHUNK, ZERO_ROWS_PER_TILE - off)
            pltpu.sync_copy(
                buf0.at[pl.ds(0, n)],
                acc.at[pl.ds(s * ZERO_ROWS_PER_TILE + off, n)],
            )
            off += n
        plsc.subcore_barrier()

        isems = (isem0, isem1)

        def fire_blk(m, slot):
            pltpu.async_copy(src_hbm.at[wid, pl.ds(m * IB, IB)], sblk.at[slot], isems[slot])
            pltpu.async_copy(dst_hbm.at[wid, pl.ds(m * IB, IB)], dblk.at[slot], isems[slot])

        def drain_blk(slot):
            pltpu.make_async_copy(src_hbm.at[wid, pl.ds(0, IB)], sblk.at[slot], isems[slot]).wait()
            pltpu.make_async_copy(dst_hbm.at[wid, pl.ds(0, IB)], dblk.at[slot], isems[slot]).wait()

        def fire_rows(buf, slot, k, sem):
            pltpu.async_copy(x_hbm.at[sblk.at[slot, k]], buf, sem)

        def drain_rows(buf, sem):
            pltpu.make_async_copy(x_hbm.at[sblk.at[0, 0]], buf, sem).wait()

        def scat(buf, slot, k):
            pltpu.sync_copy(buf, acc.at[dblk.at[slot, k]], add=True)

        def do_block(slot, boundary_drain, boundary_fire, nxt):
            for k in range(0, IB, 2):
                fire_rows(buf1, slot, k + 1, gsem1)
                drain_rows(buf0, gsem0)
                scat(buf0, slot, k)
                if k == IB - 2 and boundary_drain is not None:
                    boundary_drain()
                drain_rows(buf1, gsem1)
                scat(buf1, slot, k + 1)
                if k == IB - 2 and boundary_fire is not None:
                    boundary_fire()
                if k + 2 < IB:
                    fire_rows(buf0, slot, k + 2, gsem0)
                elif nxt is not None:
                    fire_rows(buf0, nxt, 0, gsem0)

        fire_blk(0, 0)
        fire_blk(1, 1)
        drain_blk(0)
        fire_rows(buf0, 0, 0, gsem0)

        @pl.loop(0, NBLK // 2 - 1)
        def _edges(t):
            do_block(0, lambda: drain_blk(1), lambda: fire_blk(2 * t + 2, 0), 1)
            do_block(1, lambda: drain_blk(0), lambda: fire_blk(2 * t + 3, 1), 0)

        do_block(0, lambda: drain_blk(1), None, 1)
        do_block(1, None, None, None)

        plsc.subcore_barrier()

        pltpu.sync_copy(
            acc.at[pl.ds(s * ZERO_ROWS_PER_TILE, ZERO_ROWS_PER_TILE)],
            out_hbm.at[c, pl.ds(s * ZERO_ROWS_PER_TILE, ZERO_ROWS_PER_TILE)],
        )

    return seg_sum(x, src_t, dst_t)


def _dense_layer_tc(parts, x, w_rel, b, w_root):
    def body(p_ref, x_ref, wr_ref, b_ref, wq_ref, o_ref):
        agg = p_ref[0, :N_NODES] + p_ref[1, :N_NODES]
        h = jnp.dot(agg, wr_ref[...], preferred_element_type=jnp.float32)
        h = h + jnp.dot(x_ref[...], wq_ref[...], preferred_element_type=jnp.float32)
        h = h + b_ref[...]
        o_ref[...] = jnp.maximum(h, 0.0)

    return pl.pallas_call(
        body,
        out_shape=jax.ShapeDtypeStruct((N_NODES, D), jnp.float32),
    )(parts, x, w_rel, b.reshape(1, D), w_root)


def _final_tc(parts, h, w_rel, b, w_root, batch2d, fc_w, fc_b):
    def body(p_ref, h_ref, wr_ref, b_ref, wq_ref, bt_ref, fw_ref, fb_ref, o_ref):
        agg = p_ref[0, :N_NODES] + p_ref[1, :N_NODES]
        h2 = jnp.dot(agg, wr_ref[...], preferred_element_type=jnp.float32)
        h2 = h2 + jnp.dot(h_ref[...], wq_ref[...], preferred_element_type=jnp.float32)
        h2 = jnp.maximum(h2 + b_ref[...], 0.0)
        gids = lax.broadcasted_iota(jnp.int32, (N_GRAPHS, N_NODES), 0)
        sel = (gids == bt_ref[...]).astype(jnp.float32)
        sums = jnp.dot(sel, h2, preferred_element_type=jnp.float32)
        counts = jnp.sum(sel, axis=1, keepdims=True)
        pooled = sums / jnp.maximum(counts, 1.0)
        out = jnp.dot(pooled, fw_ref[...], preferred_element_type=jnp.float32)
        o_ref[...] = out + fb_ref[...]

    return pl.pallas_call(
        body,
        out_shape=jax.ShapeDtypeStruct((N_GRAPHS, 10), jnp.float32),
    )(parts, h, w_rel, b.reshape(1, D), w_root, batch2d, fc_w, fc_b.reshape(1, 10))


def kernel(x, edge_index, batch, W1_rel, b1, W1_root, W2_rel, b2, W2_root, fc_W, fc_b):
    x = x.astype(jnp.float32)
    src = edge_index[0].astype(jnp.int32)
    dst = edge_index[1].astype(jnp.int32)

    n_pad = E_PAD - N_EDGES
    pad_ids = jnp.arange(n_pad, dtype=jnp.int32)
    pad_src = (pad_ids * 97) % N_NODES
    pad_dst = N_NODES + pad_ids % (ACC_ROWS - N_NODES)
    src_t = jnp.concatenate([src, pad_src]).reshape(NW, CPT, CHUNK)
    dst_t = jnp.concatenate([dst, pad_dst]).reshape(NW, CPT, CHUNK)

    p1 = _segment_sum_sc(x, src_t, dst_t)
    h = _dense_layer_tc(p1, x, W1_rel, b1, W1_root)
    p2 = _segment_sum_sc(h, src_t, dst_t)
    batch2d = batch.astype(jnp.int32).reshape(1, N_NODES)
    return _final_tc(p2, h, W2_rel, b2, W2_root, batch2d, fc_W, fc_b)


# speedup vs baseline: 1.0557x; 1.0557x over previous
import functools

import jax
import jax.numpy as jnp
from jax import lax
from jax.experimental import pallas as pl
from jax.experimental.pallas import tpu as pltpu
from jax.experimental.pallas import tpu_sc as plsc

N_NODES = 10000
N_EDGES = 320000
D = 128
N_GRAPHS = 64

NC = 2
NS = 16
NW = NC * NS

CHUNK = 128
IB = 8
_CPT_MIN = (N_EDGES + NW * CHUNK - 1) // (NW * CHUNK)
CPT = ((_CPT_MIN + 2 * IB - 1) // (2 * IB)) * (2 * IB)
NBLK = CPT // IB
E_PAD = NW * CPT * CHUNK

ACC_ROWS = 10240
ZERO_ROWS_PER_TILE = ACC_ROWS // NS
BR = 1024


def _segment_sum_sc(x, src_t, dst_t):
    mesh = plsc.VectorSubcoreMesh(core_axis_name="c", subcore_axis_name="s")

    @functools.partial(
        pl.kernel,
        out_type=jax.ShapeDtypeStruct((NC, ACC_ROWS, D), jnp.float32),
        mesh=mesh,
        scratch_types=[
            pltpu.VMEM_SHARED((ACC_ROWS, D), jnp.float32),
            pltpu.VMEM((2, CHUNK), jnp.int32),
            pltpu.VMEM((2, CHUNK), jnp.int32),
            pltpu.VMEM((CHUNK, D), jnp.float32),
            pltpu.VMEM((CHUNK, D), jnp.float32),
            pltpu.SemaphoreType.DMA,
            pltpu.SemaphoreType.DMA,
            pltpu.SemaphoreType.DMA,
            pltpu.SemaphoreType.DMA,
        ],
    )
    def seg_sum(x_hbm, src_hbm, dst_hbm, out_hbm, acc, sidx, didx, buf0, buf1,
                gsem0, gsem1, isem0, isem1):
        c = lax.axis_index("c")
        s = lax.axis_index("s")
        wid = c * NS + s

        zvec = jnp.zeros((16,), jnp.float32)

        @pl.loop(0, CHUNK)
        def _zero_rows(r):
            for cc in range(D // 16):
                buf0[r, pl.ds(cc * 16, 16)] = zvec

        off = 0
        while off < ZERO_ROWS_PER_TILE:
            n = min(CHUNK, ZERO_ROWS_PER_TILE - off)
            pltpu.sync_copy(
                buf0.at[pl.ds(0, n)],
                acc.at[pl.ds(s * ZERO_ROWS_PER_TILE + off, n)],
            )
            off += n
        plsc.subcore_barrier()

        isems = (isem0, isem1)

        def fire_idx(i, slot):
            pltpu.async_copy(src_hbm.at[wid, i], sidx.at[slot], isems[slot])
            pltpu.async_copy(dst_hbm.at[wid, i], didx.at[slot], isems[slot])

        def drain_idx(slot):
            pltpu.make_async_copy(src_hbm.at[wid, 0], sidx.at[slot], isems[slot]).wait()
            pltpu.make_async_copy(dst_hbm.at[wid, 0], didx.at[slot], isems[slot]).wait()

        def fire_rows(slot, buf, sem):
            pltpu.async_copy(x_hbm.at[sidx.at[slot]], buf, sem)

        def drain_rows(slot, buf, sem):
            pltpu.make_async_copy(x_hbm.at[sidx.at[slot]], buf, sem).wait()

        def scat(slot, buf):
            pltpu.sync_copy(buf, acc.at[didx.at[slot]], add=True)

        fire_idx(0, 0)
        fire_idx(1, 1)
        drain_idx(0)
        fire_rows(0, buf0, gsem0)

        @pl.loop(0, CPT // 2 - 1)
        def _edges(j):
            i = j * 2
            drain_idx(1)
            fire_rows(1, buf1, gsem1)
            drain_rows(0, buf0, gsem0)
            scat(0, buf0)
            fire_idx(i + 2, 0)
            drain_rows(1, buf1, gsem1)
            scat(1, buf1)
            fire_idx(i + 3, 1)
            drain_idx(0)
            fire_rows(0, buf0, gsem0)

        drain_idx(1)
        fire_rows(1, buf1, gsem1)
        drain_rows(0, buf0, gsem0)
        scat(0, buf0)
        drain_rows(1, buf1, gsem1)
        scat(1, buf1)

        plsc.subcore_barrier()

        pltpu.sync_copy(
            acc.at[pl.ds(s * ZERO_ROWS_PER_TILE, ZERO_ROWS_PER_TILE)],
            out_hbm.at[c, pl.ds(s * ZERO_ROWS_PER_TILE, ZERO_ROWS_PER_TILE)],
        )

    return seg_sum(x, src_t, dst_t)


def _dense_layer_tc(parts, x, w_rel, b, w_root):
    def body(p_ref, x_ref, wr_ref, b_ref, wq_ref, o_ref):
        agg = p_ref[0] + p_ref[1]
        h = jnp.dot(agg, wr_ref[...], preferred_element_type=jnp.float32)
        h = h + jnp.dot(x_ref[...], wq_ref[...], preferred_element_type=jnp.float32)
        h = h + b_ref[...]
        o_ref[...] = jnp.maximum(h, 0.0)

    return pl.pallas_call(
        body,
        grid=(ACC_ROWS // BR,),
        in_specs=[
            pl.BlockSpec((2, BR, D), lambda i: (0, i, 0)),
            pl.BlockSpec((BR, D), lambda i: (i, 0)),
            pl.BlockSpec((D, D), lambda i: (0, 0)),
            pl.BlockSpec((1, D), lambda i: (0, 0)),
            pl.BlockSpec((D, D), lambda i: (0, 0)),
        ],
        out_specs=pl.BlockSpec((BR, D), lambda i: (i, 0)),
        out_shape=jax.ShapeDtypeStruct((ACC_ROWS, D), jnp.float32),
    )(parts, x, w_rel, b.reshape(1, D), w_root)


def _final_tc(parts, h, w_rel, b, w_root, batch2d, fc_w, fc_b):
    nsteps = ACC_ROWS // BR

    def body(p_ref, h_ref, wr_ref, b_ref, wq_ref, bt_ref, fw_ref, fb_ref,
             o_ref, sums_ref, cnts_ref):
        step = pl.program_id(0)

        @pl.when(step == 0)
        def _init():
            sums_ref[...] = jnp.zeros_like(sums_ref)
            cnts_ref[...] = jnp.zeros_like(cnts_ref)

        agg = p_ref[0] + p_ref[1]
        h2 = jnp.dot(agg, wr_ref[...], preferred_element_type=jnp.float32)
        h2 = h2 + jnp.dot(h_ref[...], wq_ref[...], preferred_element_type=jnp.float32)
        h2 = jnp.maximum(h2 + b_ref[...], 0.0)
        gids = lax.broadcasted_iota(jnp.int32, (N_GRAPHS, BR), 0)
        sel = (gids == bt_ref[...]).astype(jnp.float32)
        sums_ref[...] += jnp.dot(sel, h2, preferred_element_type=jnp.float32)
        cnts_ref[...] += jnp.sum(sel, axis=1, keepdims=True)

        @pl.when(step == nsteps - 1)
        def _fin():
            pooled = sums_ref[...] / jnp.maximum(cnts_ref[...], 1.0)
            out = jnp.dot(pooled, fw_ref[...], preferred_element_type=jnp.float32)
            o_ref[...] = out + fb_ref[...]

    return pl.pallas_call(
        body,
        grid=(nsteps,),
        in_specs=[
            pl.BlockSpec((2, BR, D), lambda i: (0, i, 0)),
            pl.BlockSpec((BR, D), lambda i: (i, 0)),
            pl.BlockSpec((D, D), lambda i: (0, 0)),
            pl.BlockSpec((1, D), lambda i: (0, 0)),
            pl.BlockSpec((D, D), lambda i: (0, 0)),
            pl.BlockSpec((1, BR), lambda i: (0, i)),
            pl.BlockSpec((D, 10), lambda i: (0, 0)),
            pl.BlockSpec((1, 10), lambda i: (0, 0)),
        ],
        out_specs=pl.BlockSpec((N_GRAPHS, 10), lambda i: (0, 0)),
        out_shape=jax.ShapeDtypeStruct((N_GRAPHS, 10), jnp.float32),
        scratch_shapes=[
            pltpu.VMEM((N_GRAPHS, D), jnp.float32),
            pltpu.VMEM((N_GRAPHS, D), jnp.float32),
        ],
    )(parts, h, w_rel, b.reshape(1, D), w_root, batch2d, fc_w, fc_b.reshape(1, 10))


def kernel(x, edge_index, batch, W1_rel, b1, W1_root, W2_rel, b2, W2_root, fc_W, fc_b):
    x = x.astype(jnp.float32)
    src = edge_index[0].astype(jnp.int32)
    dst = edge_index[1].astype(jnp.int32)

    n_pad = E_PAD - N_EDGES
    pad_ids = jnp.arange(n_pad, dtype=jnp.int32)
    pad_src = (pad_ids * 97) % N_NODES
    pad_dst = N_NODES + pad_ids % (ACC_ROWS - N_NODES)
    src_t = jnp.concatenate([src, pad_src]).reshape(NW, CPT, CHUNK)
    dst_t = jnp.concatenate([dst, pad_dst]).reshape(NW, CPT, CHUNK)

    x_pad = jnp.concatenate(
        [x, jnp.zeros((ACC_ROWS - N_NODES, D), jnp.float32)])
    batch_pad = jnp.concatenate(
        [batch.astype(jnp.int32),
         jnp.full((ACC_ROWS - N_NODES,), -1, jnp.int32)]).reshape(1, ACC_ROWS)

    p1 = _segment_sum_sc(x_pad, src_t, dst_t)
    h = _dense_layer_tc(p1, x_pad, W1_rel, b1, W1_root)
    p2 = _segment_sum_sc(h, src_t, dst_t)
    return _final_tc(p2, h, W2_rel, b2, W2_root, batch_pad, fc_W, fc_b)


# speedup vs baseline: 1.0799x; 1.0229x over previous
import functools

import jax
import jax.numpy as jnp
from jax import lax
from jax.experimental import pallas as pl
from jax.experimental.pallas import tpu as pltpu
from jax.experimental.pallas import tpu_sc as plsc

N_NODES = 10000
N_EDGES = 320000
D = 128
N_GRAPHS = 64

NC = 2
NS = 16
NW = NC * NS

CHUNK = 128
IB = 8
_CPT_MIN = (N_EDGES + NW * CHUNK - 1) // (NW * CHUNK)
CPT = ((_CPT_MIN + 2 * IB - 1) // (2 * IB)) * (2 * IB)
NBLK = CPT // IB
E_PAD = NW * CPT * CHUNK

ACC_ROWS = 10240
ZERO_ROWS_PER_TILE = ACC_ROWS // NS
BR = 1024


def _segment_sum_sc(x, src_t, dst_t):
    mesh = plsc.VectorSubcoreMesh(core_axis_name="c", subcore_axis_name="s")

    @functools.partial(
        pl.kernel,
        out_type=jax.ShapeDtypeStruct((NC, ACC_ROWS, D), jnp.float32),
        mesh=mesh,
        scratch_types=[
            pltpu.VMEM_SHARED((ACC_ROWS, D), jnp.float32),
            pltpu.VMEM((2, CHUNK), jnp.int32),
            pltpu.VMEM((2, CHUNK), jnp.int32),
            pltpu.VMEM((CHUNK, D), jnp.float32),
            pltpu.VMEM((CHUNK, D), jnp.float32),
            pltpu.SemaphoreType.DMA,
            pltpu.SemaphoreType.DMA,
            pltpu.SemaphoreType.DMA,
            pltpu.SemaphoreType.DMA,
        ],
    )
    def seg_sum(x_hbm, src_hbm, dst_hbm, out_hbm, acc, sidx, didx, buf0, buf1,
                gsem0, gsem1, isem0, isem1):
        c = lax.axis_index("c")
        s = lax.axis_index("s")
        wid = c * NS + s

        zvec = jnp.zeros((16,), jnp.float32)

        @pl.loop(0, CHUNK)
        def _zero_rows(r):
            for cc in range(D // 16):
                buf0[r, pl.ds(cc * 16, 16)] = zvec

        off = 0
        while off < ZERO_ROWS_PER_TILE:
            n = min(CHUNK, ZERO_ROWS_PER_TILE - off)
            pltpu.sync_copy(
                buf0.at[pl.ds(0, n)],
                acc.at[pl.ds(s * ZERO_ROWS_PER_TILE + off, n)],
            )
            off += n
        plsc.subcore_barrier()

        isems = (isem0, isem1)

        def fire_idx(i, slot):
            pltpu.async_copy(src_hbm.at[wid, i], sidx.at[slot], isems[slot])
            pltpu.async_copy(dst_hbm.at[wid, i], didx.at[slot], isems[slot])

        def drain_idx(slot):
            pltpu.make_async_copy(src_hbm.at[wid, 0], sidx.at[slot], isems[slot]).wait()
            pltpu.make_async_copy(dst_hbm.at[wid, 0], didx.at[slot], isems[slot]).wait()

        def fire_rows(slot, buf, sem):
            pltpu.async_copy(x_hbm.at[sidx.at[slot]], buf, sem)

        def drain_rows(slot, buf, sem):
            pltpu.make_async_copy(x_hbm.at[sidx.at[slot]], buf, sem).wait()

        def scat(slot, buf):
            pltpu.sync_copy(buf, acc.at[didx.at[slot]], add=True)

        fire_idx(0, 0)
        fire_idx(1, 1)
        drain_idx(0)
        fire_rows(0, buf0, gsem0)

        @pl.loop(0, CPT // 2 - 1)
        def _edges(j):
            i = j * 2
            drain_idx(1)
            fire_rows(1, buf1, gsem1)
            drain_rows(0, buf0, gsem0)
            scat(0, buf0)
            fire_idx(i + 2, 0)
            drain_rows(1, buf1, gsem1)
            scat(1, buf1)
            fire_idx(i + 3, 1)
            drain_idx(0)
            fire_rows(0, buf0, gsem0)

        drain_idx(1)
        fire_rows(1, buf1, gsem1)
        drain_rows(0, buf0, gsem0)
        scat(0, buf0)
        drain_rows(1, buf1, gsem1)
        scat(1, buf1)

        plsc.subcore_barrier()

        pltpu.sync_copy(
            acc.at[pl.ds(s * ZERO_ROWS_PER_TILE, ZERO_ROWS_PER_TILE)],
            out_hbm.at[c, pl.ds(s * ZERO_ROWS_PER_TILE, ZERO_ROWS_PER_TILE)],
        )

    return seg_sum(x, src_t, dst_t)


def _dense_layer_tc(parts, x, w_rel, b, w_root):
    def body(p_ref, x_ref, wr_ref, b_ref, wq_ref, o_ref):
        agg = p_ref[0, :N_NODES] + p_ref[1, :N_NODES]
        h = jnp.dot(agg, wr_ref[...], preferred_element_type=jnp.float32)
        h = h + jnp.dot(x_ref[...], wq_ref[...], preferred_element_type=jnp.float32)
        h = h + b_ref[...]
        o_ref[...] = jnp.maximum(h, 0.0)

    return pl.pallas_call(
        body,
        out_shape=jax.ShapeDtypeStruct((N_NODES, D), jnp.float32),
    )(parts, x, w_rel, b.reshape(1, D), w_root)


def _final_tc(parts, h, w_rel, b, w_root, batch2d, fc_w, fc_b):
    def body(p_ref, h_ref, wr_ref, b_ref, wq_ref, bt_ref, fw_ref, fb_ref, o_ref):
        agg = p_ref[0, :N_NODES] + p_ref[1, :N_NODES]
        h2 = jnp.dot(agg, wr_ref[...], preferred_element_type=jnp.float32)
        h2 = h2 + jnp.dot(h_ref[...], wq_ref[...], preferred_element_type=jnp.float32)
        h2 = jnp.maximum(h2 + b_ref[...], 0.0)
        gids = lax.broadcasted_iota(jnp.int32, (N_GRAPHS, N_NODES), 0)
        sel = (gids == bt_ref[...]).astype(jnp.float32)
        sums = jnp.dot(sel, h2, preferred_element_type=jnp.float32)
        counts = jnp.sum(sel, axis=1, keepdims=True)
        pooled = sums / jnp.maximum(counts, 1.0)
        out = jnp.dot(pooled, fw_ref[...], preferred_element_type=jnp.float32)
        o_ref[...] = out + fb_ref[...]

    return pl.pallas_call(
        body,
        out_shape=jax.ShapeDtypeStruct((N_GRAPHS, 10), jnp.float32),
    )(parts, h, w_rel, b.reshape(1, D), w_root, batch2d, fc_w, fc_b.reshape(1, 10))


def kernel(x, edge_index, batch, W1_rel, b1, W1_root, W2_rel, b2, W2_root, fc_W, fc_b):
    x = x.astype(jnp.float32)
    src = edge_index[0].astype(jnp.int32)
    dst = edge_index[1].astype(jnp.int32)

    n_pad = E_PAD - N_EDGES
    pad_ids = jnp.arange(n_pad, dtype=jnp.int32)
    pad_src = (pad_ids * 97) % N_NODES
    pad_dst = N_NODES + pad_ids % (ACC_ROWS - N_NODES)
    src_t = jnp.concatenate([src, pad_src]).reshape(NW, CPT, CHUNK)
    dst_t = jnp.concatenate([dst, pad_dst]).reshape(NW, CPT, CHUNK)

    p1 = _segment_sum_sc(x, src_t, dst_t)
    h = _dense_layer_tc(p1, x, W1_rel, b1, W1_root)
    p2 = _segment_sum_sc(h, src_t, dst_t)
    batch2d = batch.astype(jnp.int32).reshape(1, N_NODES)
    return _final_tc(p2, h, W2_rel, b2, W2_root, batch2d, fc_W, fc_b)


# speedup vs baseline: 1.0942x; 1.0133x over previous
import functools

import jax
import jax.numpy as jnp
from jax import lax
from jax.experimental import pallas as pl
from jax.experimental.pallas import tpu as pltpu
from jax.experimental.pallas import tpu_sc as plsc

N_NODES = 10000
N_EDGES = 320000
D = 128
N_GRAPHS = 64

NC = 2
NS = 16
NW = NC * NS

CHUNK = 128
IB = 8
_CPT_MIN = (N_EDGES + NW * CHUNK - 1) // (NW * CHUNK)
CPT = ((_CPT_MIN + 2 * IB - 1) // (2 * IB)) * (2 * IB)
NBLK = CPT // IB
E_PAD = NW * CPT * CHUNK

ACC_ROWS = 10240
ZERO_ROWS_PER_TILE = ACC_ROWS // NS
BR = 1024


def _segment_sum_sc(x, src_t, dst_t):
    mesh = plsc.VectorSubcoreMesh(core_axis_name="c", subcore_axis_name="s")

    @functools.partial(
        pl.kernel,
        out_type=jax.ShapeDtypeStruct((NC, ACC_ROWS, D), jnp.float32),
        mesh=mesh,
        scratch_types=[
            pltpu.VMEM_SHARED((ACC_ROWS, D), jnp.float32),
            pltpu.VMEM((2, CHUNK), jnp.int32),
            pltpu.VMEM((2, CHUNK), jnp.int32),
            pltpu.VMEM((CHUNK, D), jnp.float32),
            pltpu.VMEM((CHUNK, D), jnp.float32),
            pltpu.SemaphoreType.DMA,
            pltpu.SemaphoreType.DMA,
            pltpu.SemaphoreType.DMA,
            pltpu.SemaphoreType.DMA,
            pltpu.SemaphoreType.DMA,
        ],
    )
    def seg_sum(x_hbm, src_hbm, dst_hbm, out_hbm, acc, sidx, didx, buf0, buf1,
                gsem0, gsem1, isem0, isem1, zsem):
        c = lax.axis_index("c")
        s = lax.axis_index("s")
        wid = c * NS + s

        isems = (isem0, isem1)

        def fire_idx(i, slot):
            pltpu.async_copy(src_hbm.at[wid, i], sidx.at[slot], isems[slot])
            pltpu.async_copy(dst_hbm.at[wid, i], didx.at[slot], isems[slot])

        def drain_idx(slot):
            pltpu.make_async_copy(src_hbm.at[wid, 0], sidx.at[slot], isems[slot]).wait()
            pltpu.make_async_copy(dst_hbm.at[wid, 0], didx.at[slot], isems[slot]).wait()

        def fire_rows(slot, buf, sem):
            pltpu.async_copy(x_hbm.at[sidx.at[slot]], buf, sem)

        def drain_rows(slot, buf, sem):
            pltpu.make_async_copy(x_hbm.at[sidx.at[slot]], buf, sem).wait()

        def scat(slot, buf):
            pltpu.sync_copy(buf, acc.at[didx.at[slot]], add=True)

        fire_idx(0, 0)
        fire_idx(1, 1)

        zvec = jnp.zeros((16,), jnp.float32)

        @pl.loop(0, CHUNK)
        def _zero_rows(r):
            for cc in range(D // 16):
                buf1[r, pl.ds(cc * 16, 16)] = zvec

        nz = ZERO_ROWS_PER_TILE // CHUNK
        for b in range(nz):
            pltpu.async_copy(
                buf1,
                acc.at[pl.ds(s * ZERO_ROWS_PER_TILE + b * CHUNK, CHUNK)],
                zsem,
            )
        drain_idx(0)
        fire_rows(0, buf0, gsem0)
        for b in range(nz):
            pltpu.make_async_copy(
                buf1,
                acc.at[pl.ds(s * ZERO_ROWS_PER_TILE + b * CHUNK, CHUNK)],
                zsem,
            ).wait()
        plsc.subcore_barrier()

        @pl.loop(0, CPT // 2 - 1)
        def _edges(j):
            i = j * 2
            drain_idx(1)
            fire_rows(1, buf1, gsem1)
            drain_rows(0, buf0, gsem0)
            scat(0, buf0)
            fire_idx(i + 2, 0)
            drain_rows(1, buf1, gsem1)
            scat(1, buf1)
            fire_idx(i + 3, 1)
            drain_idx(0)
            fire_rows(0, buf0, gsem0)

        drain_idx(1)
        fire_rows(1, buf1, gsem1)
        drain_rows(0, buf0, gsem0)
        scat(0, buf0)
        drain_rows(1, buf1, gsem1)
        scat(1, buf1)

        plsc.subcore_barrier()

        pltpu.sync_copy(
            acc.at[pl.ds(s * ZERO_ROWS_PER_TILE, ZERO_ROWS_PER_TILE)],
            out_hbm.at[c, pl.ds(s * ZERO_ROWS_PER_TILE, ZERO_ROWS_PER_TILE)],
        )

    return seg_sum(x, src_t, dst_t)


def _dense_layer_tc(parts, x, w_rel, b, w_root):
    def body(p_ref, x_ref, wr_ref, b_ref, wq_ref, o_ref):
        agg = p_ref[0, :N_NODES] + p_ref[1, :N_NODES]
        h = jnp.dot(agg, wr_ref[...], preferred_element_type=jnp.float32)
        h = h + jnp.dot(x_ref[...], wq_ref[...], preferred_element_type=jnp.float32)
        h = h + b_ref[...]
        o_ref[...] = jnp.maximum(h, 0.0)

    return pl.pallas_call(
        body,
        out_shape=jax.ShapeDtypeStruct((N_NODES, D), jnp.float32),
    )(parts, x, w_rel, b.reshape(1, D), w_root)


def _final_tc(parts, h, w_rel, b, w_root, batch2d, fc_w, fc_b):
    def body(p_ref, h_ref, wr_ref, b_ref, wq_ref, bt_ref, fw_ref, fb_ref, o_ref):
        agg = p_ref[0, :N_NODES] + p_ref[1, :N_NODES]
        h2 = jnp.dot(agg, wr_ref[...], preferred_element_type=jnp.float32)
        h2 = h2 + jnp.dot(h_ref[...], wq_ref[...], preferred_element_type=jnp.float32)
        h2 = jnp.maximum(h2 + b_ref[...], 0.0)
        gids = lax.broadcasted_iota(jnp.int32, (N_GRAPHS, N_NODES), 0)
        sel = (gids == bt_ref[...]).astype(jnp.float32)
        sums = jnp.dot(sel, h2, preferred_element_type=jnp.float32)
        counts = jnp.sum(sel, axis=1, keepdims=True)
        pooled = sums / jnp.maximum(counts, 1.0)
        out = jnp.dot(pooled, fw_ref[...], preferred_element_type=jnp.float32)
        o_ref[...] = out + fb_ref[...]

    return pl.pallas_call(
        body,
        out_shape=jax.ShapeDtypeStruct((N_GRAPHS, 10), jnp.float32),
    )(parts, h, w_rel, b.reshape(1, D), w_root, batch2d, fc_w, fc_b.reshape(1, 10))


def kernel(x, edge_index, batch, W1_rel, b1, W1_root, W2_rel, b2, W2_root, fc_W, fc_b):
    x = x.astype(jnp.float32)
    src = edge_index[0].astype(jnp.int32)
    dst = edge_index[1].astype(jnp.int32)

    n_pad = E_PAD - N_EDGES
    pad_ids = jnp.arange(n_pad, dtype=jnp.int32)
    pad_src = (pad_ids * 97) % N_NODES
    pad_dst = N_NODES + pad_ids % (ACC_ROWS - N_NODES)
    src_t = jnp.concatenate([src, pad_src]).reshape(NW, CPT, CHUNK)
    dst_t = jnp.concatenate([dst, pad_dst]).reshape(NW, CPT, CHUNK)

    p1 = _segment_sum_sc(x, src_t, dst_t)
    h = _dense_layer_tc(p1, x, W1_rel, b1, W1_root)
    p2 = _segment_sum_sc(h, src_t, dst_t)
    batch2d = batch.astype(jnp.int32).reshape(1, N_NODES)
    return _final_tc(p2, h, W2_rel, b2, W2_root, batch2d, fc_W, fc_b)


# speedup vs baseline: 1.0992x; 1.0046x over previous
import functools

import jax
import jax.numpy as jnp
from jax import lax
from jax.experimental import pallas as pl
from jax.experimental.pallas import tpu as pltpu
from jax.experimental.pallas import tpu_sc as plsc

N_NODES = 10000
N_EDGES = 320000
D = 128
N_GRAPHS = 64

NC = 2
NS = 16
NW = NC * NS

CHUNK = 128
IB = 8
_CPT_MIN = (N_EDGES + NW * CHUNK - 1) // (NW * CHUNK)
CPT = ((_CPT_MIN + 2 * IB - 1) // (2 * IB)) * (2 * IB)
NBLK = CPT // IB
E_PAD = NW * CPT * CHUNK

ACC_ROWS = 10240
ZERO_ROWS_PER_TILE = ACC_ROWS // NS
BR = 1024


def _segment_sum_sc(x, src_t, dst_t):
    mesh = plsc.VectorSubcoreMesh(core_axis_name="c", subcore_axis_name="s")

    @functools.partial(
        pl.kernel,
        out_type=jax.ShapeDtypeStruct((NC, ACC_ROWS, D), jnp.float32),
        mesh=mesh,
        scratch_types=[
            pltpu.VMEM_SHARED((ACC_ROWS, D), jnp.float32),
            pltpu.VMEM((2, CHUNK), jnp.int32),
            pltpu.VMEM((2, CHUNK), jnp.int32),
            pltpu.VMEM((CHUNK, D), jnp.float32),
            pltpu.VMEM((CHUNK, D), jnp.float32),
            pltpu.SemaphoreType.DMA,
            pltpu.SemaphoreType.DMA,
            pltpu.SemaphoreType.DMA,
            pltpu.SemaphoreType.DMA,
            pltpu.SemaphoreType.DMA,
        ],
    )
    def seg_sum(x_hbm, src_hbm, dst_hbm, out_hbm, acc, sidx, didx, buf0, buf1,
                gsem0, gsem1, isem0, isem1, zsem):
        c = lax.axis_index("c")
        s = lax.axis_index("s")
        wid = c * NS + s

        isems = (isem0, isem1)

        def fire_idx(i, slot):
            pltpu.async_copy(src_hbm.at[wid, i], sidx.at[slot], isems[slot])
            pltpu.async_copy(dst_hbm.at[wid, i], didx.at[slot], isems[slot])

        def drain_idx(slot):
            pltpu.make_async_copy(src_hbm.at[wid, 0], sidx.at[slot], isems[slot]).wait()
            pltpu.make_async_copy(dst_hbm.at[wid, 0], didx.at[slot], isems[slot]).wait()

        def fire_rows(slot, buf, sem):
            pltpu.async_copy(x_hbm.at[sidx.at[slot]], buf, sem)

        def drain_rows(slot, buf, sem):
            pltpu.make_async_copy(x_hbm.at[sidx.at[slot]], buf, sem).wait()

        def scat(slot, buf):
            pltpu.sync_copy(buf, acc.at[didx.at[slot]], add=True)

        fire_idx(0, 0)
        fire_idx(1, 1)

        zvec = jnp.zeros((16,), jnp.float32)

        @pl.loop(0, CHUNK)
        def _zero_rows(r):
            for cc in range(D // 16):
                buf1[r, pl.ds(cc * 16, 16)] = zvec

        nz = ZERO_ROWS_PER_TILE // CHUNK
        for b in range(nz):
            pltpu.async_copy(
                buf1,
                acc.at[pl.ds(s * ZERO_ROWS_PER_TILE + b * CHUNK, CHUNK)],
                zsem,
            )
        drain_idx(0)
        fire_rows(0, buf0, gsem0)
        for b in range(nz):
            pltpu.make_async_copy(
                buf1,
                acc.at[pl.ds(s * ZERO_ROWS_PER_TILE + b * CHUNK, CHUNK)],
                zsem,
            ).wait()
        plsc.subcore_barrier()

        @pl.loop(0, CPT // 2 - 1)
        def _edges(j):
            i = j * 2
            drain_idx(1)
            fire_rows(1, buf1, gsem1)
            drain_rows(0, buf0, gsem0)
            scat(0, buf0)
            fire_idx(i + 2, 0)
            drain_rows(1, buf1, gsem1)
            scat(1, buf1)
            fire_idx(i + 3, 1)
            drain_idx(0)
            fire_rows(0, buf0, gsem0)

        drain_idx(1)
        fire_rows(1, buf1, gsem1)
        drain_rows(0, buf0, gsem0)
        scat(0, buf0)
        drain_rows(1, buf1, gsem1)
        scat(1, buf1)

        plsc.subcore_barrier()

        pltpu.sync_copy(
            acc.at[pl.ds(s * ZERO_ROWS_PER_TILE, ZERO_ROWS_PER_TILE)],
            out_hbm.at[c, pl.ds(s * ZERO_ROWS_PER_TILE, ZERO_ROWS_PER_TILE)],
        )

    return seg_sum(x, src_t, dst_t)


def _root_tc(x, w_root):
    def body(x_ref, wq_ref, o_ref):
        o_ref[...] = jnp.dot(x_ref[...], wq_ref[...],
                             preferred_element_type=jnp.float32)

    return pl.pallas_call(
        body,
        out_shape=jax.ShapeDtypeStruct((N_NODES, D), jnp.float32),
    )(x, w_root)


def _dense_layer_tc(parts, root, w_rel, b):
    def body(p_ref, r_ref, wr_ref, b_ref, o_ref):
        agg = p_ref[0, :N_NODES] + p_ref[1, :N_NODES]
        h = jnp.dot(agg, wr_ref[...], preferred_element_type=jnp.float32)
        h = h + r_ref[...] + b_ref[...]
        o_ref[...] = jnp.maximum(h, 0.0)

    return pl.pallas_call(
        body,
        out_shape=jax.ShapeDtypeStruct((N_NODES, D), jnp.float32),
    )(parts, root, w_rel, b.reshape(1, D))


def _final_tc(parts, root, w_rel, b, batch2d, fc_w, fc_b):
    def body(p_ref, r_ref, wr_ref, b_ref, bt_ref, fw_ref, fb_ref, o_ref):
        agg = p_ref[0, :N_NODES] + p_ref[1, :N_NODES]
        h2 = jnp.dot(agg, wr_ref[...], preferred_element_type=jnp.float32)
        h2 = jnp.maximum(h2 + r_ref[...] + b_ref[...], 0.0)
        gids = lax.broadcasted_iota(jnp.int32, (N_GRAPHS, N_NODES), 0)
        sel = (gids == bt_ref[...]).astype(jnp.float32)
        sums = jnp.dot(sel, h2, preferred_element_type=jnp.float32)
        counts = jnp.sum(sel, axis=1, keepdims=True)
        pooled = sums / jnp.maximum(counts, 1.0)
        out = jnp.dot(pooled, fw_ref[...], preferred_element_type=jnp.float32)
        o_ref[...] = out + fb_ref[...]

    return pl.pallas_call(
        body,
        out_shape=jax.ShapeDtypeStruct((N_GRAPHS, 10), jnp.float32),
    )(parts, root, w_rel, b.reshape(1, D), batch2d, fc_w, fc_b.reshape(1, 10))


def kernel(x, edge_index, batch, W1_rel, b1, W1_root, W2_rel, b2, W2_root, fc_W, fc_b):
    x = x.astype(jnp.float32)
    src = edge_index[0].astype(jnp.int32)
    dst = edge_index[1].astype(jnp.int32)

    n_pad = E_PAD - N_EDGES
    pad_ids = jnp.arange(n_pad, dtype=jnp.int32)
    pad_src = (pad_ids * 97) % N_NODES
    pad_dst = N_NODES + pad_ids % (ACC_ROWS - N_NODES)
    src_t = jnp.concatenate([src, pad_src]).reshape(NW, CPT, CHUNK)
    dst_t = jnp.concatenate([dst, pad_dst]).reshape(NW, CPT, CHUNK)

    p1 = _segment_sum_sc(x, src_t, dst_t)
    root1 = _root_tc(x, W1_root)
    h = _dense_layer_tc(p1, root1, W1_rel, b1)
    p2 = _segment_sum_sc(h, src_t, dst_t)
    root2 = _root_tc(h, W2_root)
    batch2d = batch.astype(jnp.int32).reshape(1, N_NODES)
    return _final_tc(p2, root2, W2_rel, b2, batch2d, fc_W, fc_b)
